# baseline (speedup 1.0000x reference)
import functools

import jax
import jax.numpy as jnp
from jax import lax
from jax.experimental import pallas as pl
from jax.experimental.pallas import tpu as pltpu
from jax.experimental.pallas import tpu_sc as plsc

NC = 2
NS = 16
EDGE_CHUNK = 400
NBUF = 2
LANES = 16


def _pad_rows(n):
  return ((n + NS * 8 - 1) // (NS * 8)) * NS * 8


def _static_chunks(total, chunk):
  out, off = [], 0
  while off < total:
    sz = min(chunk, total - off)
    out.append((off, sz))
    off += sz
  return out


def _sc_agg_body(with_cnt, n, e, hd, *refs):
  if with_cnt:
    (feat, src2, dst, agg_out, cnt_out) = refs[:5]
    (ones_v, acc_sh, cnt_sh) = refs[5 + 3 * NBUF:5 + 3 * NBUF + 3]
    rest = refs[5:5 + 3 * NBUF] + refs[5 + 3 * NBUF + 3:]
  else:
    (feat, src2, dst, agg_out) = refs[:4]
    acc_sh = refs[4 + 3 * NBUF]
    ones_v = cnt_sh = None
    rest = refs[4:4 + 3 * NBUF] + refs[4 + 3 * NBUF + 1:]
  sidxs = rest[0:NBUF]
  didxs = rest[NBUF:2 * NBUF]
  rows = rest[2 * NBUF:3 * NBUF]
  gsems = rest[3 * NBUF:4 * NBUF]
  ssems = rest[4 * NBUF:5 * NBUF]
  csem = rest[5 * NBUF] if with_cnt else None
  rows_a = rows[0]
  c = lax.axis_index("c")
  s = lax.axis_index("s")
  n_pad = _pad_rows(n)
  rows_per_tile = n_pad // NS
  ch = EDGE_CHUNK

  def zrow(i, _):
    for j in range(hd // LANES):
      rows_a[i, pl.ds(j * LANES, LANES)] = jnp.zeros((LANES,), jnp.float32)
    return 0
  lax.fori_loop(0, ch, zrow, 0)

  base = s * rows_per_tile
  for off, sz in _static_chunks(rows_per_tile, ch):
    pltpu.sync_copy(rows_a.at[pl.ds(0, sz)], acc_sh.at[pl.ds(base + off, sz)])

  if with_cnt:
    def zone(i, _):
      ones_v[i, :] = jnp.zeros((LANES,), jnp.float32)
      return 0
    lax.fori_loop(0, ch, zone, 0)
    for off, sz in _static_chunks(rows_per_tile, ch):
      pltpu.sync_copy(ones_v.at[pl.ds(0, sz)],
                      cnt_sh.at[pl.ds(base + off, sz)])
    def sone(i, _):
      ones_v[i, :] = jnp.ones((LANES,), jnp.float32)
      return 0
    lax.fori_loop(0, ch, sone, 0)

  plsc.subcore_barrier()

  e_per_tile = e // NS
  cpt = e_per_tile // ch
  n_rounds = cpt // NBUF
  tail = cpt - n_rounds * NBUF
  ebase = s * e_per_tile
  sbase = c * e + ebase

  def drain(buf, sem):
    pltpu.make_async_copy(feat.at[pl.ds(0, ch)], buf, sem).wait()

  def idx_drain(buf, sem):
    pltpu.make_async_copy(dst.at[pl.ds(0, ch)], buf, sem).wait()

  def body(p, _):
    base_r = NBUF * p * ch
    for q in range(NBUF):
      off = base_r + q * ch

      @pl.when(p > 0)
      def _drain_scat(ro=rows[q], ss=ssems[q]):
        drain(ro, ss)
      pltpu.async_copy(src2.at[pl.ds(sbase + off, ch)], sidxs[q], gsems[q])
      pltpu.async_copy(dst.at[pl.ds(ebase + off, ch)], didxs[q], gsems[q])
    for q in range(NBUF):
      idx_drain(sidxs[q], gsems[q])
      idx_drain(didxs[q], gsems[q])
      pltpu.async_copy(feat.at[sidxs[q]], rows[q], gsems[q])
    for q in range(NBUF):
      drain(rows[q], gsems[q])
      pltpu.async_copy(rows[q], acc_sh.at[didxs[q]], ssems[q], add=True)
      if with_cnt:
        @pl.when(c == ((NBUF * p + q) % 2))
        def _cnt(di=didxs[q]):
          pltpu.async_copy(ones_v, cnt_sh.at[di], csem, add=True)
    return 0
  lax.fori_loop(0, n_rounds, body, 0)
  for t in range(tail):
    off = n_rounds * NBUF * ch + t * ch
    drain(rows[t], ssems[t])
    pltpu.sync_copy(src2.at[pl.ds(sbase + off, ch)], sidxs[t])
    pltpu.sync_copy(dst.at[pl.ds(ebase + off, ch)], didxs[t])
    pltpu.async_copy(feat.at[sidxs[t]], rows[t], gsems[t])
  for t in range(tail):
    drain(rows[t], gsems[t])
    pltpu.async_copy(rows[t], acc_sh.at[didxs[t]], ssems[t], add=True)
    if with_cnt:
      jj = n_rounds * NBUF + t
      @pl.when(c == (jj % 2))
      def _cnt_t(di=didxs[t]):
        pltpu.async_copy(ones_v, cnt_sh.at[di], csem, add=True)
  for q in range(NBUF):
    drain(rows[q], ssems[q])
  if with_cnt:
    assert cpt % 2 == 0

    def cnt_drain(i, _):
      pltpu.make_async_copy(cnt_out.at[0, pl.ds(0, ch)], ones_v, csem).wait()
      return 0
    lax.fori_loop(0, cpt // 2, cnt_drain, 0)

  plsc.subcore_barrier()

  pltpu.sync_copy(acc_sh.at[pl.ds(base, rows_per_tile)],
                  agg_out.at[c, pl.ds(base, rows_per_tile)])
  if with_cnt:
    pltpu.sync_copy(cnt_sh.at[pl.ds(base, rows_per_tile)],
                    cnt_out.at[c, pl.ds(base, rows_per_tile)])


def _make_sc_agg(with_cnt, n, e, hd):
  mesh = plsc.VectorSubcoreMesh(core_axis_name="c", subcore_axis_name="s")
  n_pad = _pad_rows(n)
  out_type = [jax.ShapeDtypeStruct((NC, n_pad, hd), jnp.float32)]
  scratch = [pltpu.VMEM((EDGE_CHUNK,), jnp.int32)] * NBUF
  scratch += [pltpu.VMEM((EDGE_CHUNK,), jnp.int32)] * NBUF
  scratch += [pltpu.VMEM((EDGE_CHUNK, hd), jnp.float32)] * NBUF
  if with_cnt:
    out_type.append(jax.ShapeDtypeStruct((NC, n_pad, LANES), jnp.float32))
    scratch.append(pltpu.VMEM((EDGE_CHUNK, LANES), jnp.float32))
  scratch.append(pltpu.VMEM_SHARED((n_pad, hd), jnp.float32))
  if with_cnt:
    scratch.append(pltpu.VMEM_SHARED((n_pad, LANES), jnp.float32))
  scratch += [pltpu.SemaphoreType.DMA] * (2 * NBUF)
  if with_cnt:
    scratch.append(pltpu.SemaphoreType.DMA)
  return pl.kernel(
      functools.partial(_sc_agg_body, with_cnt, n, e, hd),
      out_type=tuple(out_type),
      mesh=mesh,
      scratch_types=tuple(scratch),
      compiler_params=pltpu.CompilerParams(use_tc_tiling_on_sc=False),
  )


def _split_halves(h, n, hd):
  return jnp.stack([h[:, :hd], h[:, hd:]])


def _tc_xr_body(halves, x_ref, wrT_ref, bl_ref, o_ref):
  if halves:
    x = jnp.concatenate([x_ref[0], x_ref[1]], axis=1)
  else:
    x = x_ref[...]
  o_ref[...] = (jnp.dot(x, wrT_ref[...], preferred_element_type=jnp.float32)
                + bl_ref[...])


def _make_tc_xr(halves, n, d, block):
  grid = n // block
  hd = d // 2
  xspec = (pl.BlockSpec((NC, block, hd), lambda i: (0, i, 0)) if halves
           else pl.BlockSpec((block, d), lambda i: (i, 0)))
  return pl.pallas_call(
      functools.partial(_tc_xr_body, halves),
      grid=(grid,),
      in_specs=[
          xspec,
          pl.BlockSpec((d, d), lambda i: (0, 0)),
          pl.BlockSpec((1, d), lambda i: (0, 0)),
      ],
      out_specs=pl.BlockSpec((block, d), lambda i: (i, 0)),
      out_shape=jax.ShapeDtypeStruct((n, d), jnp.float32),
  )


def _tc_layer_body(relu, agg_ref, cntp_ref, xr_ref, wlT_ref, o_ref):
  agg = jnp.concatenate([agg_ref[0], agg_ref[1]], axis=1)
  cnt = cntp_ref[0, :, 0] + cntp_ref[1, :, 0]
  rc = 1.0 / jnp.maximum(cnt, 1.0)
  mean = agg * rc[:, None]
  h = (jnp.dot(mean, wlT_ref[...], preferred_element_type=jnp.float32)
       + xr_ref[...])
  h = jnp.maximum(h, 0.0) if relu else h
  hd = h.shape[1] // 2
  o_ref[0] = h[:, :hd]
  o_ref[1] = h[:, hd:]


def _make_tc_layer(relu, n, d, block):
  grid = n // block
  hd = d // 2
  return pl.pallas_call(
      functools.partial(_tc_layer_body, relu),
      grid=(grid,),
      in_specs=[
          pl.BlockSpec((NC, block, hd), lambda i: (0, i, 0)),
          pl.BlockSpec((NC, block, LANES), lambda i: (0, i, 0)),
          pl.BlockSpec((block, d), lambda i: (i, 0)),
          pl.BlockSpec((d, d), lambda i: (0, 0)),
      ],
      out_specs=pl.BlockSpec((NC, block, hd), lambda i: (0, i, 0)),
      out_shape=jax.ShapeDtypeStruct((NC, n, hd), jnp.float32),
  )


def _tc_final_body(g, block, agg_ref, cntp_ref, xr_ref, wlT_ref,
                   batch_ref, gf_ref, wgT_ref, bg_ref, woaT_ref,
                   wobT_ref, wocT_ref, bo_ref, o_ref, sum_acc, cnt_acc,
                   max_acc):
  i = pl.program_id(0)
  nblocks = pl.num_programs(0)

  @pl.when(i == 0)
  def _init():
    sum_acc[...] = jnp.zeros_like(sum_acc)
    cnt_acc[...] = jnp.zeros_like(cnt_acc)
    max_acc[...] = jnp.full_like(max_acc, -jnp.inf)

  agg = jnp.concatenate([agg_ref[0], agg_ref[1]], axis=1)
  cnt = cntp_ref[0, :, 0] + cntp_ref[1, :, 0]
  rc = 1.0 / jnp.maximum(cnt, 1.0)
  mean = agg * rc[:, None]
  h = (jnp.dot(mean, wlT_ref[...], preferred_element_type=jnp.float32)
       + xr_ref[...])

  bcol = batch_ref[...]
  gids = lax.broadcasted_iota(jnp.int32, (block, g), 1)
  onehot = (bcol == gids).astype(jnp.float32)
  sum_acc[...] += lax.dot_general(
      onehot, h, (((0,), (0,)), ((), ())),
      preferred_element_type=jnp.float32)
  cnt_acc[...] += lax.dot_general(
      onehot, jnp.ones_like(h), (((0,), (0,)), ((), ())),
      preferred_element_type=jnp.float32)

  d = h.shape[1]
  bb = jnp.broadcast_to(bcol, (block, d))
  hm = h
  k = 1
  while k < block:
    bs = pltpu.roll(bb, k, 0)
    hs = pltpu.roll(hm, k, 0)
    hm = jnp.maximum(hm, jnp.where(bb == bs, hs, -jnp.inf))
    k *= 2
  bnext = pltpu.roll(bb, block - 1, 0)
  rows = lax.broadcasted_iota(jnp.int32, (block, d), 0)
  is_last = (bb != bnext) | (rows == block - 1)
  lastcol = jnp.where(is_last[:, :1], 1.0, 0.0)
  oh_last = onehot * lastcol
  picked = lax.dot_general(oh_last, hm, (((0,), (0,)), ((), ())),
                           preferred_element_type=jnp.float32)
  pres = lax.dot_general(oh_last, jnp.ones_like(h), (((0,), (0,)), ((), ())),
                         preferred_element_type=jnp.float32)
  max_acc[...] = jnp.maximum(
      max_acc[...], jnp.where(pres > 0.5, picked, -jnp.inf))

  @pl.when(i == nblocks - 1)
  def _finish():
    mean_pool = sum_acc[...] * (1.0 / jnp.maximum(cnt_acc[...], 1.0))
    gft = (jnp.dot(gf_ref[...], wgT_ref[...],
                   preferred_element_type=jnp.float32) + bg_ref[...])
    logits = (jnp.dot(mean_pool, woaT_ref[...],
                      preferred_element_type=jnp.float32)
              + jnp.dot(max_acc[...], wobT_ref[...],
                        preferred_element_type=jnp.float32)
              + jnp.dot(gft, wocT_ref[...],
                        preferred_element_type=jnp.float32)
              + bo_ref[...])
    m = jnp.max(logits, axis=1, keepdims=True)
    lse = m + jnp.log(jnp.sum(jnp.exp(logits - m), axis=1, keepdims=True))
    o_ref[...] = logits - lse


def _make_tc_final(g, n, d, gf, block):
  grid = n // block
  hd = d // 2
  return pl.pallas_call(
      functools.partial(_tc_final_body, g, block),
      grid=(grid,),
      in_specs=[
          pl.BlockSpec((NC, block, hd), lambda i: (0, i, 0)),
          pl.BlockSpec((NC, block, LANES), lambda i: (0, i, 0)),
          pl.BlockSpec((block, d), lambda i: (i, 0)),
          pl.BlockSpec((d, d), lambda i: (0, 0)),
          pl.BlockSpec((block, 1), lambda i: (i, 0)),
          pl.BlockSpec((g, gf), lambda i: (0, 0)),
          pl.BlockSpec((gf, d), lambda i: (0, 0)),
          pl.BlockSpec((1, d), lambda i: (0, 0)),
          pl.BlockSpec((d, d), lambda i: (0, 0)),
          pl.BlockSpec((d, d), lambda i: (0, 0)),
          pl.BlockSpec((d, d), lambda i: (0, 0)),
          pl.BlockSpec((1, d), lambda i: (0, 0)),
      ],
      out_specs=pl.BlockSpec((g, d), lambda i: (0, 0)),
      out_shape=jax.ShapeDtypeStruct((g, d), jnp.float32),
      scratch_shapes=[
          pltpu.VMEM((g, d), jnp.float32),
          pltpu.VMEM((g, d), jnp.float32),
          pltpu.VMEM((g, d), jnp.float32),
      ],
  )


def kernel(x, edges_idx, batch_idx, g_features, Wl0, bl0, Wr0, Wl1, bl1, Wr1,
           Wg, bg, Wo, bo):
  n, d = x.shape
  e = edges_idx.shape[1]
  g, gf = g_features.shape
  hd = d // 2
  block = 2000

  src = edges_idx[0]
  dst = edges_idx[1]
  batch_col = batch_idx.reshape(n, 1)

  woT = Wo.T
  zpad = jnp.zeros((d, d - 2), jnp.float32)
  woaT = jnp.concatenate([woT[:d], zpad], axis=1)
  wobT = jnp.concatenate([woT[d:2 * d], zpad], axis=1)
  wocT = jnp.concatenate([woT[2 * d:], zpad], axis=1)
  bo_p = jnp.concatenate(
      [bo, jnp.full((d - 2,), -1e30, jnp.float32)]).reshape(1, d)

  sc_agg0 = _make_sc_agg(True, n, e, hd)
  sc_agg1 = _make_sc_agg(False, n, e, hd)
  tc_xr0 = _make_tc_xr(False, n, d, block)
  tc_xr1 = _make_tc_xr(True, n, d, block)
  tc_layer0 = _make_tc_layer(True, n, d, block)
  tc_final = _make_tc_final(g, n, d, gf, block)

  src2 = jnp.concatenate([src, src + n])
  x_halves = _split_halves(x, n, hd)
  xr0 = tc_xr0(x, Wr0.T, bl0.reshape(1, d))
  agg0, cntp = sc_agg0(x_halves.reshape(2 * n, hd), src2, dst)
  h0 = tc_layer0(agg0, cntp, xr0, Wl0.T)
  xr1 = tc_xr1(h0, Wr1.T, bl1.reshape(1, d))
  (agg1,) = sc_agg1(h0.reshape(2 * n, hd), src2, dst)
  out = tc_final(agg1, cntp, xr1, Wl1.T, batch_col,
                 g_features, Wg.T, bg.reshape(1, d), woaT, wobT, wocT, bo_p)
  return out[:, :2]


# speedup vs baseline: 1.1204x; 1.1204x over previous
import functools

import jax
import jax.numpy as jnp
from jax import lax
from jax.experimental import pallas as pl
from jax.experimental.pallas import tpu as pltpu
from jax.experimental.pallas import tpu_sc as plsc

NC = 2
NS = 16
EDGE_CHUNK = 400
NBUF = 2
LANES = 16


def _pad_rows(n):
  return ((n + NS * 8 - 1) // (NS * 8)) * NS * 8


def _static_chunks(total, chunk):
  out, off = [], 0
  while off < total:
    sz = min(chunk, total - off)
    out.append((off, sz))
    off += sz
  return out


def _sc_agg_body(with_cnt, n, e, hd, *refs):
  if with_cnt:
    (feat, src2, dst, agg_out, cnt_out) = refs[:5]
    (ones_v, acc_sh, cnt_sh) = refs[5 + 3 * NBUF:5 + 3 * NBUF + 3]
    rest = refs[5:5 + 3 * NBUF] + refs[5 + 3 * NBUF + 3:]
  else:
    (feat, src2, dst, agg_out) = refs[:4]
    acc_sh = refs[4 + 3 * NBUF]
    ones_v = cnt_sh = None
    rest = refs[4:4 + 3 * NBUF] + refs[4 + 3 * NBUF + 1:]
  sidxs = rest[0:NBUF]
  didxs = rest[NBUF:2 * NBUF]
  rows = rest[2 * NBUF:3 * NBUF]
  gsems = rest[3 * NBUF:4 * NBUF]
  ssems = rest[4 * NBUF:5 * NBUF]
  csem = rest[5 * NBUF] if with_cnt else None
  rows_a = rows[0]
  c = lax.axis_index("c")
  s = lax.axis_index("s")
  n_pad = _pad_rows(n)
  rows_per_tile = n_pad // NS
  ch = EDGE_CHUNK

  def zrow(i, _):
    for j in range(hd // LANES):
      rows_a[i, pl.ds(j * LANES, LANES)] = jnp.zeros((LANES,), jnp.float32)
    return 0
  lax.fori_loop(0, ch, zrow, 0)

  base = s * rows_per_tile
  for off, sz in _static_chunks(rows_per_tile, ch):
    pltpu.sync_copy(rows_a.at[pl.ds(0, sz)], acc_sh.at[pl.ds(base + off, sz)])

  if with_cnt:
    def zone(i, _):
      ones_v[i, :] = jnp.zeros((LANES,), jnp.float32)
      return 0
    lax.fori_loop(0, ch, zone, 0)
    for off, sz in _static_chunks(rows_per_tile, ch):
      pltpu.sync_copy(ones_v.at[pl.ds(0, sz)],
                      cnt_sh.at[pl.ds(base + off, sz)])
    def sone(i, _):
      ones_v[i, :] = jnp.ones((LANES,), jnp.float32)
      return 0
    lax.fori_loop(0, ch, sone, 0)

  plsc.subcore_barrier()

  e_per_tile = e // NS
  cpt = e_per_tile // ch
  n_rounds = cpt // NBUF
  tail = cpt - n_rounds * NBUF
  ebase = s * e_per_tile
  sbase = c * e + ebase

  def drain(buf, sem):
    pltpu.make_async_copy(feat.at[pl.ds(0, ch)], buf, sem).wait()

  def body(p, _):
    base_r = NBUF * p * ch
    for q in range(NBUF):
      off = base_r + q * ch

      @pl.when(p > 0)
      def _drain_scat(ro=rows[q], ss=ssems[q]):
        drain(ro, ss)
      pltpu.sync_copy(src2.at[pl.ds(sbase + off, ch)], sidxs[q])
      pltpu.sync_copy(dst.at[pl.ds(ebase + off, ch)], didxs[q])
      pltpu.async_copy(feat.at[sidxs[q]], rows[q], gsems[q])
    for q in range(NBUF):
      drain(rows[q], gsems[q])
      pltpu.async_copy(rows[q], acc_sh.at[didxs[q]], ssems[q], add=True)
      if with_cnt:
        @pl.when(c == ((NBUF * p + q) % 2))
        def _cnt(di=didxs[q]):
          pltpu.async_copy(ones_v, cnt_sh.at[di], csem, add=True)
    return 0
  lax.fori_loop(0, n_rounds, body, 0)
  for t in range(tail):
    off = n_rounds * NBUF * ch + t * ch
    drain(rows[t], ssems[t])
    pltpu.sync_copy(src2.at[pl.ds(sbase + off, ch)], sidxs[t])
    pltpu.sync_copy(dst.at[pl.ds(ebase + off, ch)], didxs[t])
    pltpu.async_copy(feat.at[sidxs[t]], rows[t], gsems[t])
  for t in range(tail):
    drain(rows[t], gsems[t])
    pltpu.async_copy(rows[t], acc_sh.at[didxs[t]], ssems[t], add=True)
    if with_cnt:
      jj = n_rounds * NBUF + t
      @pl.when(c == (jj % 2))
      def _cnt_t(di=didxs[t]):
        pltpu.async_copy(ones_v, cnt_sh.at[di], csem, add=True)
  for q in range(NBUF):
    drain(rows[q], ssems[q])
  if with_cnt:
    assert cpt % 2 == 0

    def cnt_drain(i, _):
      pltpu.make_async_copy(cnt_out.at[0, pl.ds(0, ch)], ones_v, csem).wait()
      return 0
    lax.fori_loop(0, cpt // 2, cnt_drain, 0)

  plsc.subcore_barrier()

  pltpu.sync_copy(acc_sh.at[pl.ds(base, rows_per_tile)],
                  agg_out.at[c, pl.ds(base, rows_per_tile)])
  if with_cnt:
    pltpu.sync_copy(cnt_sh.at[pl.ds(base, rows_per_tile)],
                    cnt_out.at[c, pl.ds(base, rows_per_tile)])


def _make_sc_agg(with_cnt, n, e, hd):
  mesh = plsc.VectorSubcoreMesh(core_axis_name="c", subcore_axis_name="s")
  n_pad = _pad_rows(n)
  out_type = [jax.ShapeDtypeStruct((NC, n_pad, hd), jnp.float32)]
  scratch = [pltpu.VMEM((EDGE_CHUNK,), jnp.int32)] * NBUF
  scratch += [pltpu.VMEM((EDGE_CHUNK,), jnp.int32)] * NBUF
  scratch += [pltpu.VMEM((EDGE_CHUNK, hd), jnp.float32)] * NBUF
  if with_cnt:
    out_type.append(jax.ShapeDtypeStruct((NC, n_pad, LANES), jnp.float32))
    scratch.append(pltpu.VMEM((EDGE_CHUNK, LANES), jnp.float32))
  scratch.append(pltpu.VMEM_SHARED((n_pad, hd), jnp.float32))
  if with_cnt:
    scratch.append(pltpu.VMEM_SHARED((n_pad, LANES), jnp.float32))
  scratch += [pltpu.SemaphoreType.DMA] * (2 * NBUF)
  if with_cnt:
    scratch.append(pltpu.SemaphoreType.DMA)
  return pl.kernel(
      functools.partial(_sc_agg_body, with_cnt, n, e, hd),
      out_type=tuple(out_type),
      mesh=mesh,
      scratch_types=tuple(scratch),
      compiler_params=pltpu.CompilerParams(use_tc_tiling_on_sc=False),
  )


def _split_halves(h, n, hd):
  return jnp.stack([h[:, :hd], h[:, hd:]])


def _tc_xr_body(halves, x_ref, wrT_ref, bl_ref, o_ref):
  if halves:
    x = jnp.concatenate([x_ref[0], x_ref[1]], axis=1)
  else:
    x = x_ref[...]
  o_ref[...] = (jnp.dot(x, wrT_ref[...], preferred_element_type=jnp.float32)
                + bl_ref[...])


def _make_tc_xr(halves, n, d, block):
  grid = n // block
  hd = d // 2
  xspec = (pl.BlockSpec((NC, block, hd), lambda i: (0, i, 0)) if halves
           else pl.BlockSpec((block, d), lambda i: (i, 0)))
  return pl.pallas_call(
      functools.partial(_tc_xr_body, halves),
      grid=(grid,),
      in_specs=[
          xspec,
          pl.BlockSpec((d, d), lambda i: (0, 0)),
          pl.BlockSpec((1, d), lambda i: (0, 0)),
      ],
      out_specs=pl.BlockSpec((block, d), lambda i: (i, 0)),
      out_shape=jax.ShapeDtypeStruct((n, d), jnp.float32),
  )


def _tc_layer_body(relu, agg_ref, cntp_ref, xr_ref, wlT_ref, o_ref):
  agg = jnp.concatenate([agg_ref[0], agg_ref[1]], axis=1)
  cnt = cntp_ref[0, :, 0] + cntp_ref[1, :, 0]
  rc = 1.0 / jnp.maximum(cnt, 1.0)
  mean = agg * rc[:, None]
  h = (jnp.dot(mean, wlT_ref[...], preferred_element_type=jnp.float32)
       + xr_ref[...])
  h = jnp.maximum(h, 0.0) if relu else h
  hd = h.shape[1] // 2
  o_ref[0] = h[:, :hd]
  o_ref[1] = h[:, hd:]


def _make_tc_layer(relu, n, d, block):
  grid = n // block
  hd = d // 2
  return pl.pallas_call(
      functools.partial(_tc_layer_body, relu),
      grid=(grid,),
      in_specs=[
          pl.BlockSpec((NC, block, hd), lambda i: (0, i, 0)),
          pl.BlockSpec((NC, block, LANES), lambda i: (0, i, 0)),
          pl.BlockSpec((block, d), lambda i: (i, 0)),
          pl.BlockSpec((d, d), lambda i: (0, 0)),
      ],
      out_specs=pl.BlockSpec((NC, block, hd), lambda i: (0, i, 0)),
      out_shape=jax.ShapeDtypeStruct((NC, n, hd), jnp.float32),
  )


def _tc_final_body(g, block, agg_ref, cntp_ref, xr_ref, wlT_ref,
                   batch_ref, gf_ref, wgT_ref, bg_ref, woaT_ref,
                   wobT_ref, wocT_ref, bo_ref, o_ref, sum_acc, cnt_acc,
                   max_acc):
  i = pl.program_id(0)
  nblocks = pl.num_programs(0)

  @pl.when(i == 0)
  def _init():
    sum_acc[...] = jnp.zeros_like(sum_acc)
    cnt_acc[...] = jnp.zeros_like(cnt_acc)
    max_acc[...] = jnp.full_like(max_acc, -jnp.inf)

  agg = jnp.concatenate([agg_ref[0], agg_ref[1]], axis=1)
  cnt = cntp_ref[0, :, 0] + cntp_ref[1, :, 0]
  rc = 1.0 / jnp.maximum(cnt, 1.0)
  mean = agg * rc[:, None]
  h = (jnp.dot(mean, wlT_ref[...], preferred_element_type=jnp.float32)
       + xr_ref[...])

  bcol = batch_ref[...]
  gids = lax.broadcasted_iota(jnp.int32, (block, g), 1)
  onehot = (bcol == gids).astype(jnp.float32)
  sum_acc[...] += lax.dot_general(
      onehot, h, (((0,), (0,)), ((), ())),
      preferred_element_type=jnp.float32)
  cnt_acc[...] += lax.dot_general(
      onehot, jnp.ones_like(h), (((0,), (0,)), ((), ())),
      preferred_element_type=jnp.float32)

  d = h.shape[1]
  bb = jnp.broadcast_to(bcol, (block, d))
  hm = h
  k = 1
  while k < block:
    bs = pltpu.roll(bb, k, 0)
    hs = pltpu.roll(hm, k, 0)
    hm = jnp.maximum(hm, jnp.where(bb == bs, hs, -jnp.inf))
    k *= 2
  bnext = pltpu.roll(bb, block - 1, 0)
  rows = lax.broadcasted_iota(jnp.int32, (block, d), 0)
  is_last = (bb != bnext) | (rows == block - 1)
  lastcol = jnp.where(is_last[:, :1], 1.0, 0.0)
  oh_last = onehot * lastcol
  picked = lax.dot_general(oh_last, hm, (((0,), (0,)), ((), ())),
                           preferred_element_type=jnp.float32)
  pres = lax.dot_general(oh_last, jnp.ones_like(h), (((0,), (0,)), ((), ())),
                         preferred_element_type=jnp.float32)
  max_acc[...] = jnp.maximum(
      max_acc[...], jnp.where(pres > 0.5, picked, -jnp.inf))

  @pl.when(i == nblocks - 1)
  def _finish():
    mean_pool = sum_acc[...] * (1.0 / jnp.maximum(cnt_acc[...], 1.0))
    gft = (jnp.dot(gf_ref[...], wgT_ref[...],
                   preferred_element_type=jnp.float32) + bg_ref[...])
    logits = (jnp.dot(mean_pool, woaT_ref[...],
                      preferred_element_type=jnp.float32)
              + jnp.dot(max_acc[...], wobT_ref[...],
                        preferred_element_type=jnp.float32)
              + jnp.dot(gft, wocT_ref[...],
                        preferred_element_type=jnp.float32)
              + bo_ref[...])
    m = jnp.max(logits, axis=1, keepdims=True)
    lse = m + jnp.log(jnp.sum(jnp.exp(logits - m), axis=1, keepdims=True))
    o_ref[...] = logits - lse


def _make_tc_final(g, n, d, gf, block):
  grid = n // block
  hd = d // 2
  return pl.pallas_call(
      functools.partial(_tc_final_body, g, block),
      grid=(grid,),
      in_specs=[
          pl.BlockSpec((NC, block, hd), lambda i: (0, i, 0)),
          pl.BlockSpec((NC, block, LANES), lambda i: (0, i, 0)),
          pl.BlockSpec((block, d), lambda i: (i, 0)),
          pl.BlockSpec((d, d), lambda i: (0, 0)),
          pl.BlockSpec((block, 1), lambda i: (i, 0)),
          pl.BlockSpec((g, gf), lambda i: (0, 0)),
          pl.BlockSpec((gf, d), lambda i: (0, 0)),
          pl.BlockSpec((1, d), lambda i: (0, 0)),
          pl.BlockSpec((d, d), lambda i: (0, 0)),
          pl.BlockSpec((d, d), lambda i: (0, 0)),
          pl.BlockSpec((d, d), lambda i: (0, 0)),
          pl.BlockSpec((1, d), lambda i: (0, 0)),
      ],
      out_specs=pl.BlockSpec((g, d), lambda i: (0, 0)),
      out_shape=jax.ShapeDtypeStruct((g, d), jnp.float32),
      scratch_shapes=[
          pltpu.VMEM((g, d), jnp.float32),
          pltpu.VMEM((g, d), jnp.float32),
          pltpu.VMEM((g, d), jnp.float32),
      ],
  )


def kernel(x, edges_idx, batch_idx, g_features, Wl0, bl0, Wr0, Wl1, bl1, Wr1,
           Wg, bg, Wo, bo):
  n, d = x.shape
  e = edges_idx.shape[1]
  g, gf = g_features.shape
  hd = d // 2
  block = 2000

  src = edges_idx[0]
  dst = edges_idx[1]
  batch_col = batch_idx.reshape(n, 1)

  woT = Wo.T
  zpad = jnp.zeros((d, d - 2), jnp.float32)
  woaT = jnp.concatenate([woT[:d], zpad], axis=1)
  wobT = jnp.concatenate([woT[d:2 * d], zpad], axis=1)
  wocT = jnp.concatenate([woT[2 * d:], zpad], axis=1)
  bo_p = jnp.concatenate(
      [bo, jnp.full((d - 2,), -1e30, jnp.float32)]).reshape(1, d)

  sc_agg0 = _make_sc_agg(True, n, e, hd)
  sc_agg1 = _make_sc_agg(False, n, e, hd)
  tc_xr0 = _make_tc_xr(False, n, d, block)
  tc_xr1 = _make_tc_xr(True, n, d, block)
  tc_layer0 = _make_tc_layer(True, n, d, block)
  tc_final = _make_tc_final(g, n, d, gf, block)

  src2 = jnp.concatenate([src, src + n])
  x_halves = _split_halves(x, n, hd)
  xr0 = tc_xr0(x, Wr0.T, bl0.reshape(1, d))
  agg0, cntp = sc_agg0(x_halves.reshape(2 * n, hd), src2, dst)
  h0 = tc_layer0(agg0, cntp, xr0, Wl0.T)
  xr1 = tc_xr1(h0, Wr1.T, bl1.reshape(1, d))
  (agg1,) = sc_agg1(h0.reshape(2 * n, hd), src2, dst)
  out = tc_final(agg1, cntp, xr1, Wl1.T, batch_col,
                 g_features, Wg.T, bg.reshape(1, d), woaT, wobT, wocT, bo_p)
  return out[:, :2]
